# baseline (speedup 1.0000x reference)
import jax
import jax.numpy as jnp
from jax import lax
from jax.experimental import pallas as pl
from jax.experimental.pallas import tpu as pltpu
from jax.experimental.pallas import tpu_sc as plsc

N = 16384
C = 1000

_info = plsc.get_sparse_core_info()
_NC = _info.num_cores
_NS = _info.num_subcores
_L = _info.num_lanes
_NW = _NC * _NS
_ROWS_PER_W = N // _NW
_CHUNK = 32
_NCHUNK = _ROWS_PER_W // _CHUNK
_PW = 128


def _one_hot_body(x_hbm, out_hbm, x_v, zbuf, ones_v, p0, p1, p2, p3, sem):
    wid = lax.axis_index("s") * _NC + lax.axis_index("c")
    row0 = wid * _ROWS_PER_W

    pltpu.sync_copy(x_hbm.at[pl.ds(row0, _ROWS_PER_W)], x_v)

    zeros = jnp.zeros((_L,), jnp.int32)

    def _zero(i, _):
        for k in range(8):
            zbuf[pl.ds(i * 8 * _L + k * _L, _L)] = zeros
        return 0

    lax.fori_loop(0, _CHUNK * C // (8 * _L), _zero, 0)

    one = jnp.ones((_L,), jnp.int32)
    iota = lax.iota(jnp.int32, _L)
    pos_refs = (p0, p1, p2, p3)
    for j in range(_ROWS_PER_W // _PW):
        for k in range(_PW // _L):
            i = j * _PW + k * _L
            xv = x_v[pl.ds(i, _L)]
            pos_refs[j][pl.ds(k * _L, _L)] = (row0 + i + iota) * C + xv
    for k in range(_PW // _L):
        ones_v[pl.ds(k * _L, _L)] = one

    copies = []
    for c in range(_NCHUNK):
        dst = out_hbm.at[pl.ds((row0 + c * _CHUNK) * C, _CHUNK * C)]
        copies.append(pltpu.async_copy(zbuf, dst, sem))
    for cp in copies:
        cp.wait()

    for j in range(_ROWS_PER_W // _PW):
        pltpu.sync_copy(ones_v, out_hbm.at[pos_refs[j]])


_one_hot = pl.kernel(
    _one_hot_body,
    out_type=jax.ShapeDtypeStruct((N * C,), jnp.int32),
    mesh=plsc.VectorSubcoreMesh(core_axis_name="c", subcore_axis_name="s"),
    scratch_types=[
        pltpu.VMEM((_ROWS_PER_W,), jnp.int32),
        pltpu.VMEM((_CHUNK * C,), jnp.int32),
        pltpu.VMEM((_PW,), jnp.int32),
        pltpu.VMEM((_PW,), jnp.int32),
        pltpu.VMEM((_PW,), jnp.int32),
        pltpu.VMEM((_PW,), jnp.int32),
        pltpu.VMEM((_PW,), jnp.int32),
        pltpu.SemaphoreType.DMA,
    ],
    compiler_params=pltpu.CompilerParams(needs_layout_passes=False),
)


@jax.jit
def kernel(x):
    return _one_hot(x).reshape(N, C)


# speedup vs baseline: 1.6711x; 1.6711x over previous
import jax
import jax.numpy as jnp
from jax import lax
from jax.experimental import pallas as pl
from jax.experimental.pallas import tpu as pltpu
from jax.experimental.pallas import tpu_sc as plsc

N = 16384
C = 1000

_info = plsc.get_sparse_core_info()
_NC = _info.num_cores
_NS = _info.num_subcores
_L = _info.num_lanes
_NW = _NC * _NS
_ROWS_PER_W = N // _NW
_CHUNK = 32
_NCHUNK = _ROWS_PER_W // _CHUNK


def _one_hot_body(x_hbm, out_hbm, x_v, buf0, buf1, sem0, sem1):
    wid = lax.axis_index("s") * _NC + lax.axis_index("c")
    row0 = wid * _ROWS_PER_W

    pltpu.sync_copy(x_hbm.at[pl.ds(row0, _ROWS_PER_W)], x_v)

    zeros = jnp.zeros((_L,), jnp.int32)

    def _zero_row(r, _):
        def _zero_col(k, _):
            buf0[r, pl.ds(k * _L, _L)] = zeros
            buf1[r, pl.ds(k * _L, _L)] = zeros
            return 0

        lax.fori_loop(0, C // _L, _zero_col, 0)
        buf0[r, pl.ds(C - _L, _L)] = zeros
        buf1[r, pl.ds(C - _L, _L)] = zeros
        return 0

    lax.fori_loop(0, _CHUNK, _zero_row, 0)

    iota = lax.iota(jnp.int32, _L)
    ones = jnp.ones((_L,), jnp.int32)
    bufs = (buf0, buf1)
    sems = (sem0, sem1)
    copies = [None, None]

    for c in range(_NCHUNK):
        b = c % 2
        if c >= 2:
            copies[b].wait()
            for g in range(_CHUNK // _L):
                xv = x_v[pl.ds((c - 2) * _CHUNK + g * _L, _L)]
                plsc.store_scatter(bufs[b], [g * _L + iota, xv], zeros)
        for g in range(_CHUNK // _L):
            xv = x_v[pl.ds(c * _CHUNK + g * _L, _L)]
            plsc.store_scatter(bufs[b], [g * _L + iota, xv], ones)
        dst = out_hbm.at[pl.ds(row0 + c * _CHUNK, _CHUNK), :]
        copies[b] = pltpu.async_copy(bufs[b], dst, sems[b])

    copies[0].wait()
    copies[1].wait()


_one_hot = pl.kernel(
    _one_hot_body,
    out_type=jax.ShapeDtypeStruct((N, C), jnp.int32),
    mesh=plsc.VectorSubcoreMesh(core_axis_name="c", subcore_axis_name="s"),
    scratch_types=[
        pltpu.VMEM((_ROWS_PER_W,), jnp.int32),
        pltpu.VMEM((_CHUNK, C), jnp.int32),
        pltpu.VMEM((_CHUNK, C), jnp.int32),
        pltpu.SemaphoreType.DMA,
        pltpu.SemaphoreType.DMA,
    ],
    compiler_params=pltpu.CompilerParams(needs_layout_passes=False),
)


@jax.jit
def kernel(x):
    return _one_hot(x)


# speedup vs baseline: 1.7668x; 1.0572x over previous
import jax
import jax.numpy as jnp
from jax import lax
from jax.experimental import pallas as pl
from jax.experimental.pallas import tpu as pltpu
from jax.experimental.pallas import tpu_sc as plsc

N = 16384
C = 1000

_info = plsc.get_sparse_core_info()
_NC = _info.num_cores
_NS = _info.num_subcores
_L = _info.num_lanes
_NW = _NC * _NS
_ROWS_PER_W = N // _NW
_CHUNK = 32
_NCHUNK = _ROWS_PER_W // _CHUNK


def _one_hot_body(x_hbm, out_hbm, x_v, buf0, buf1, sem0, sem1):
    wid = lax.axis_index("s") * _NC + lax.axis_index("c")
    row0 = wid * _ROWS_PER_W

    pltpu.sync_copy(x_hbm.at[pl.ds(row0, _ROWS_PER_W)], x_v)

    zeros = jnp.zeros((_L,), jnp.int32)

    def _zero_row(r, _):
        def _zero_col(k, _):
            buf0[r, pl.ds(k * _L, _L)] = zeros
            buf1[r, pl.ds(k * _L, _L)] = zeros
            return 0

        lax.fori_loop(0, C // _L, _zero_col, 0)
        buf0[r, pl.ds(C - _L, _L)] = zeros
        buf1[r, pl.ds(C - _L, _L)] = zeros
        return 0

    lax.fori_loop(0, _CHUNK, _zero_row, 0)

    iota = lax.iota(jnp.int32, _L)
    ones = jnp.ones((_L,), jnp.int32)
    bufs = (buf0, buf1)
    sems = (sem0, sem1)
    copies = [None, None]

    for c in range(_NCHUNK):
        b = c % 2
        if c >= 2:
            copies[b].wait()
            for g in range(_CHUNK // _L):
                xv = x_v[pl.ds((c - 2) * _CHUNK + g * _L, _L)]
                plsc.store_scatter(bufs[b], [g * _L + iota, xv], zeros)
        for g in range(_CHUNK // _L):
            xv = x_v[pl.ds(c * _CHUNK + g * _L, _L)]
            plsc.store_scatter(bufs[b], [g * _L + iota, xv], ones)
        dst = out_hbm.at[pl.ds(row0 + c * _CHUNK, _CHUNK), :]
        copies[b] = pltpu.async_copy(bufs[b], dst, sems[b])

    copies[0].wait()
    copies[1].wait()


_one_hot = pl.kernel(
    _one_hot_body,
    out_type=jax.ShapeDtypeStruct((N, C), jnp.int32),
    mesh=plsc.VectorSubcoreMesh(core_axis_name="c", subcore_axis_name="s"),
    scratch_types=[
        pltpu.VMEM((_ROWS_PER_W,), jnp.int32),
        pltpu.VMEM((_CHUNK, C), jnp.int32),
        pltpu.VMEM((_CHUNK, C), jnp.int32),
        pltpu.SemaphoreType.DMA,
        pltpu.SemaphoreType.DMA,
    ],
    compiler_params=pltpu.CompilerParams(
        needs_layout_passes=False, use_tc_tiling_on_sc=True),
)


@jax.jit
def kernel(x):
    return _one_hot(x)


# speedup vs baseline: 2.0571x; 1.1643x over previous
import jax
import jax.numpy as jnp
from jax import lax
from jax.experimental import pallas as pl
from jax.experimental.pallas import tpu as pltpu

N = 16384
C = 1000
_BR = 512
_G = N // _BR


def _tc_body(x_ref, out_ref):
    xb = x_ref[0, 0, :]
    col = lax.broadcasted_iota(jnp.int32, (_BR, C), 1)
    out_ref[...] = (col == xb[:, None]).astype(jnp.int32)


_tc_one_hot = pl.pallas_call(
    _tc_body,
    grid=(_G,),
    in_specs=[pl.BlockSpec((1, 1, _BR), lambda i: (i, 0, 0))],
    out_specs=pl.BlockSpec((_BR, C), lambda i: (i, 0)),
    out_shape=jax.ShapeDtypeStruct((N, C), jnp.int32),
)


@jax.jit
def kernel(x):
    return _tc_one_hot(x.reshape(_G, 1, _BR))


# speedup vs baseline: 3.4249x; 1.6649x over previous
import functools

import jax
import jax.numpy as jnp
from jax import lax
from jax.experimental import pallas as pl
from jax.experimental.pallas import tpu as pltpu
from jax.experimental.pallas import tpu_sc as plsc

N = 16384
C = 1000

_info = plsc.get_sparse_core_info()
_NC = _info.num_cores
_NS = _info.num_subcores
_L = _info.num_lanes
_NW = _NC * _NS
_STRIPE = N // _NW
_BAND = 104
_CHUNKS = [(i * _BAND, _BAND) for i in range(C // _BAND)] + [
    ((C // _BAND) * _BAND, C - (C // _BAND) * _BAND)]


def _one_hot_t_body(x_hbm, out_hbm, x_v, buf0, buf1, sem0, sem1):
    wid = lax.axis_index("s") * _NC + lax.axis_index("c")
    col0 = wid * _STRIPE

    pltpu.sync_copy(x_hbm.at[pl.ds(col0, _STRIPE)], x_v)

    zeros = jnp.zeros((_L,), jnp.int32)

    def _zero_row(r, _):
        for k in range(_STRIPE // _L):
            buf0[r, pl.ds(k * _L, _L)] = zeros
            buf1[r, pl.ds(k * _L, _L)] = zeros
        return 0

    lax.fori_loop(0, _BAND, _zero_row, 0)

    iota = lax.iota(jnp.int32, _L)
    ones = jnp.ones((_L,), jnp.int32)
    bufs = (buf0, buf1)
    sems = (sem0, sem1)
    copies = [None, None]

    def _scatter(buf, cls0, ncls, vals):
        for g in range(_STRIPE // _L):
            xv = x_v[pl.ds(g * _L, _L)]
            m = (xv >= cls0) & (xv < cls0 + ncls)
            plsc.store_scatter(buf, [xv - cls0, g * _L + iota], vals, mask=m)

    for c, (cls0, ncls) in enumerate(_CHUNKS):
        b = c % 2
        if c >= 2:
            copies[b].wait()
            pcls0, pncls = _CHUNKS[c - 2]
            _scatter(bufs[b], pcls0, pncls, zeros)
        _scatter(bufs[b], cls0, ncls, ones)
        dst = out_hbm.at[pl.ds(cls0, ncls), pl.ds(col0, _STRIPE)]
        copies[b] = pltpu.async_copy(bufs[b].at[pl.ds(0, ncls), :], dst,
                                     sems[b])

    copies[0].wait()
    copies[1].wait()


_one_hot_t = pl.kernel(
    _one_hot_t_body,
    out_type=jax.ShapeDtypeStruct((C, N), jnp.int32),
    mesh=plsc.VectorSubcoreMesh(core_axis_name="c", subcore_axis_name="s"),
    scratch_types=[
        pltpu.VMEM((_STRIPE,), jnp.int32),
        pltpu.VMEM((_BAND, _STRIPE), jnp.int32),
        pltpu.VMEM((_BAND, _STRIPE), jnp.int32),
        pltpu.SemaphoreType.DMA,
        pltpu.SemaphoreType.DMA,
    ],
    compiler_params=pltpu.CompilerParams(
        needs_layout_passes=False, use_tc_tiling_on_sc=True),
)


@jax.jit
def kernel(x):
    return _one_hot_t(x).T
